# baseline (speedup 1.0000x reference)
import functools

import jax
import jax.numpy as jnp
from jax import lax
from jax.experimental import pallas as pl
from jax.experimental.pallas import tpu as pltpu

_TOPK = 100
_LANE = 128
_BR = 8192
_NEG = float("-inf")
_BIG = 2**30


def _matvec_kernel(qs_ref, h_ref, o1_ref, o2_ref, *, n_valid):
    g = pl.program_id(0)
    s = lax.dot_general(
        qs_ref[...], h_ref[...],
        (((1,), (1,)), ((), ())),
        preferred_element_type=jnp.float32,
    )
    pos = g * _BR + lax.broadcasted_iota(jnp.int32, (8, _BR), 1)
    s = jnp.where(pos < n_valid, s, _NEG)
    for i in range(_BR // _LANE):
        o1_ref[i:i + 1, :] = s[0:1, i * _LANE:(i + 1) * _LANE]
        o2_ref[i:i + 1, :] = s[1:2, i * _LANE:(i + 1) * _LANE]


def _topk_one(s_ref, cid_ref, cand_ref, gidx_ref, n_chunkrows):
    c_rows = n_chunkrows
    cb = c_rows // _LANE
    sv = s_ref[...]
    cm0 = jnp.max(sv.reshape(cb, _LANE, _LANE), axis=2)
    cflat = (lax.broadcasted_iota(jnp.int32, (cb, _LANE), 0) * _LANE
             + lax.broadcasted_iota(jnp.int32, (cb, _LANE), 1))
    lane1 = lax.broadcasted_iota(jnp.int32, (1, _LANE), 1)

    cm = cm0
    for i in range(_TOPK):
        m = jnp.max(cm, keepdims=True)
        c = jnp.min(jnp.where(cm == m, cflat, _BIG), keepdims=True)
        cid_ref[0, i] = c[0, 0]
        cm = jnp.where(cflat == c, _NEG, cm)

    for i in range(_TOPK):
        c_i = cid_ref[0, i]
        cand_ref[i:i + 1, :] = s_ref[pl.ds(c_i, 1), :]
        gidx_ref[i:i + 1, :] = c_i * _LANE + lane1
    pad_v = jnp.full((1, _LANE), _NEG, jnp.float32)
    pad_g = jnp.full((1, _LANE), _BIG, jnp.int32)
    for i in range(_TOPK, _LANE):
        cand_ref[i:i + 1, :] = pad_v
        gidx_ref[i:i + 1, :] = pad_g

    cv = cand_ref[...]
    gv = gidx_ref[...]
    outv = jnp.full((1, _LANE), _NEG, jnp.float32)
    outi = jnp.zeros((1, _LANE), jnp.int32)
    for i in range(_TOPK):
        m = jnp.max(cv, keepdims=True)
        p = jnp.min(jnp.where(cv == m, gv, _BIG), keepdims=True)
        outv = jnp.where(lane1 == i, m, outv)
        outi = jnp.where(lane1 == i, p, outi)
        cv = jnp.where((cv == m) & (gv == p), _NEG, cv)
    return outv, outi


def _select_kernel(s1_ref, s2_ref, v1_ref, i1_ref, v2_ref, i2_ref, loss_ref,
                   cid_ref, cand_ref, gidx_ref, *, n_chunkrows):
    v1, i1 = _topk_one(s1_ref, cid_ref, cand_ref, gidx_ref, n_chunkrows)
    v2, i2 = _topk_one(s2_ref, cid_ref, cand_ref, gidx_ref, n_chunkrows)
    v1_ref[...] = v1
    i1_ref[...] = i1
    v2_ref[...] = v2
    i2_ref[...] = i2
    lane1 = lax.broadcasted_iota(jnp.int32, (1, _LANE), 1)
    mask = lane1 < _TOPK
    e1 = jnp.sum(jnp.where(mask, jnp.exp(v1), 0.0))
    e2 = jnp.sum(jnp.where(mask, jnp.exp(v2), 0.0))
    numerator = e1 * e2
    denominator = e1 * e2
    loss_ref[...] = jnp.reshape(-jnp.log(numerator / denominator), (1, 1))


def kernel(q1, q2, H, k):
    del k
    n_rows, d = H.shape
    nb = 2 * pl.cdiv(n_rows, 2 * _BR)
    c_rows = nb * _BR // _LANE
    nbh = pl.cdiv(n_rows, _BR)
    rpb = _BR // _LANE

    qs = jnp.zeros((8, d), jnp.float32).at[0].set(q1[0]).at[1].set(q2[0])

    s1, s2 = pl.pallas_call(
        functools.partial(_matvec_kernel, n_valid=n_rows),
        grid=(nb,),
        in_specs=[
            pl.BlockSpec((8, d), lambda g: (0, 0)),
            pl.BlockSpec((_BR, d), lambda g: (jnp.minimum(g, nbh - 1), 0)),
        ],
        out_specs=[
            pl.BlockSpec((rpb, _LANE), lambda g: (g, 0)),
            pl.BlockSpec((rpb, _LANE), lambda g: (g, 0)),
        ],
        out_shape=[
            jax.ShapeDtypeStruct((c_rows, _LANE), jnp.float32),
            jax.ShapeDtypeStruct((c_rows, _LANE), jnp.float32),
        ],
    )(qs, H)

    v1, i1, v2, i2, loss = pl.pallas_call(
        functools.partial(_select_kernel, n_chunkrows=c_rows),
        out_shape=[
            jax.ShapeDtypeStruct((1, _LANE), jnp.float32),
            jax.ShapeDtypeStruct((1, _LANE), jnp.int32),
            jax.ShapeDtypeStruct((1, _LANE), jnp.float32),
            jax.ShapeDtypeStruct((1, _LANE), jnp.int32),
            jax.ShapeDtypeStruct((1, 1), jnp.float32),
        ],
        scratch_shapes=[
            pltpu.SMEM((1, _LANE), jnp.int32),
            pltpu.VMEM((_LANE, _LANE), jnp.float32),
            pltpu.VMEM((_LANE, _LANE), jnp.int32),
        ],
    )(s1, s2)

    return (loss[0, 0], v1[0, :_TOPK], i1[0, :_TOPK],
            v2[0, :_TOPK], i2[0, :_TOPK])


# speedup vs baseline: 1.3863x; 1.3863x over previous
import functools

import jax
import jax.numpy as jnp
from jax import lax
from jax.experimental import pallas as pl
from jax.experimental.pallas import tpu as pltpu

_TOPK = 100
_LANE = 128
_BR = 8192
_SENT = -3.4028235e38
_BIG = 2**30


def _matvec_kernel(qs_ref, h_ref, o1_ref, o2_ref, *, n_valid):
    g = pl.program_id(0)
    s = lax.dot_general(
        qs_ref[...], h_ref[...],
        (((1,), (1,)), ((), ())),
        preferred_element_type=jnp.float32,
    )
    pos = g * _BR + lax.broadcasted_iota(jnp.int32, (8, _BR), 1)
    s = jnp.where(pos < n_valid, s, _SENT)
    for i in range(_BR // _LANE):
        o1_ref[i:i + 1, :] = s[0:1, i * _LANE:(i + 1) * _LANE]
        o2_ref[i:i + 1, :] = s[1:2, i * _LANE:(i + 1) * _LANE]


def _lane1():
    return lax.broadcasted_iota(jnp.int32, (1, _LANE), 1)


def _cx_stage(v, x, j, want_larger, left):
    pv = jnp.where(left, pltpu.roll(v, _LANE - j, 1), pltpu.roll(v, j, 1))
    px = jnp.where(left, pltpu.roll(x, _LANE - j, 1), pltpu.roll(x, j, 1))
    gt = (v > pv) | ((v == pv) & (x < px))
    take_self = want_larger == gt
    return jnp.where(take_self, v, pv), jnp.where(take_self, x, px)


def _sort_row(v, x, descending):
    lane = _lane1()
    for k in (2, 4, 8, 16, 32, 64, 128):
        j = k // 2
        while j >= 1:
            left = (lane & j) == 0
            asc = (lane & k) == 0
            want_larger = jnp.logical_xor(left, asc)
            if descending:
                want_larger = jnp.logical_not(want_larger)
            v, x = _cx_stage(v, x, j, want_larger, left)
            j //= 2
    return v, x


def _merge_desc(mv, mi, nv, ni):
    gt = (mv > nv) | ((mv == nv) & (mi < ni))
    v = jnp.where(gt, mv, nv)
    x = jnp.where(gt, mi, ni)
    lane = _lane1()
    for j in (64, 32, 16, 8, 4, 2, 1):
        left = (lane & j) == 0
        v, x = _cx_stage(v, x, j, left, left)
    return v, x


def _round(src_val, row_iota, lane):
    rmax = jnp.max(src_val, axis=0, keepdims=True)
    rarg = jnp.min(jnp.where(src_val == rmax, row_iota, _BIG),
                   axis=0, keepdims=True)
    killed = jnp.where(row_iota == rarg, _SENT, src_val)
    gid = rarg * _LANE + lane
    return rmax, gid, killed


def _topk_one(s_ref, scr_ref, c_rows):
    lane = _lane1()
    riota = lax.broadcasted_iota(jnp.int32, (c_rows, _LANE), 0)

    rmax, gid, killed = _round(s_ref[...], riota, lane)
    scr_ref[...] = killed
    mv, mi = _sort_row(rmax, gid, descending=True)
    ub = jnp.max(rmax, keepdims=True)

    def cond(carry):
        mv, _, ub = carry
        return jnp.sum((mv > ub).astype(jnp.int32)) < _TOPK

    def body(carry):
        mv, mi, _ = carry
        rmax, gid, killed = _round(scr_ref[...], riota, lane)
        scr_ref[...] = killed
        nv, ni = _sort_row(rmax, gid, descending=False)
        mv, mi = _merge_desc(mv, mi, nv, ni)
        return mv, mi, jnp.max(rmax, keepdims=True)

    mv, mi, _ = lax.while_loop(cond, body, (mv, mi, ub))
    return mv, mi


def _select_kernel(s1_ref, s2_ref, v1_ref, i1_ref, v2_ref, i2_ref, loss_ref,
                   scr_ref, *, n_chunkrows):
    v1, i1 = _topk_one(s1_ref, scr_ref, n_chunkrows)
    v2, i2 = _topk_one(s2_ref, scr_ref, n_chunkrows)
    v1_ref[...] = v1
    i1_ref[...] = i1
    v2_ref[...] = v2
    i2_ref[...] = i2
    mask = _lane1() < _TOPK
    e1 = jnp.sum(jnp.where(mask, jnp.exp(v1), 0.0))
    e2 = jnp.sum(jnp.where(mask, jnp.exp(v2), 0.0))
    numerator = e1 * e2
    denominator = e1 * e2
    loss_ref[...] = jnp.reshape(-jnp.log(numerator / denominator), (1, 1))


def kernel(q1, q2, H, k):
    del k
    n_rows, d = H.shape
    nb = 2 * pl.cdiv(n_rows, 2 * _BR)
    c_rows = nb * _BR // _LANE
    nbh = pl.cdiv(n_rows, _BR)
    rpb = _BR // _LANE

    qs = jnp.zeros((8, d), jnp.float32).at[0].set(q1[0]).at[1].set(q2[0])

    s1, s2 = pl.pallas_call(
        functools.partial(_matvec_kernel, n_valid=n_rows),
        grid=(nb,),
        in_specs=[
            pl.BlockSpec((8, d), lambda g: (0, 0)),
            pl.BlockSpec((_BR, d), lambda g: (jnp.minimum(g, nbh - 1), 0)),
        ],
        out_specs=[
            pl.BlockSpec((rpb, _LANE), lambda g: (g, 0)),
            pl.BlockSpec((rpb, _LANE), lambda g: (g, 0)),
        ],
        out_shape=[
            jax.ShapeDtypeStruct((c_rows, _LANE), jnp.float32),
            jax.ShapeDtypeStruct((c_rows, _LANE), jnp.float32),
        ],
    )(qs, H)

    v1, i1, v2, i2, loss = pl.pallas_call(
        functools.partial(_select_kernel, n_chunkrows=c_rows),
        out_shape=[
            jax.ShapeDtypeStruct((1, _LANE), jnp.float32),
            jax.ShapeDtypeStruct((1, _LANE), jnp.int32),
            jax.ShapeDtypeStruct((1, _LANE), jnp.float32),
            jax.ShapeDtypeStruct((1, _LANE), jnp.int32),
            jax.ShapeDtypeStruct((1, 1), jnp.float32),
        ],
        scratch_shapes=[
            pltpu.VMEM((c_rows, _LANE), jnp.float32),
        ],
    )(s1, s2)

    return (loss[0, 0], v1[0, :_TOPK], i1[0, :_TOPK],
            v2[0, :_TOPK], i2[0, :_TOPK])


# speedup vs baseline: 1.6041x; 1.1571x over previous
import functools

import jax
import jax.numpy as jnp
from jax import lax
from jax.experimental import pallas as pl
from jax.experimental.pallas import tpu as pltpu

_TOPK = 100
_LANE = 128
_BR = 16384
_SENT = -3.4028235e38
_BIG = 2**30


def _matvec_kernel(qs_ref, h_ref, o1_ref, o2_ref, *, n_valid):
    g = pl.program_id(0)
    s = lax.dot_general(
        qs_ref[...], h_ref[...],
        (((1,), (1,)), ((), ())),
        preferred_element_type=jnp.float32,
    )
    pos = g * _BR + lax.broadcasted_iota(jnp.int32, (8, _BR), 1)
    s = jnp.where(pos < n_valid, s, _SENT)
    for i in range(_BR // _LANE):
        o1_ref[i:i + 1, :] = s[0:1, i * _LANE:(i + 1) * _LANE]
        o2_ref[i:i + 1, :] = s[1:2, i * _LANE:(i + 1) * _LANE]


def _lane1():
    return lax.broadcasted_iota(jnp.int32, (1, _LANE), 1)


def _cx_stage(v, x, j, want_larger, left):
    pv = jnp.where(left, pltpu.roll(v, _LANE - j, 1), pltpu.roll(v, j, 1))
    px = jnp.where(left, pltpu.roll(x, _LANE - j, 1), pltpu.roll(x, j, 1))
    gt = (v > pv) | ((v == pv) & (x < px))
    take_self = want_larger == gt
    return jnp.where(take_self, v, pv), jnp.where(take_self, x, px)


def _sort_row(v, x, descending):
    lane = _lane1()
    for k in (2, 4, 8, 16, 32, 64, 128):
        j = k // 2
        while j >= 1:
            left = (lane & j) == 0
            asc = (lane & k) == 0
            want_larger = jnp.logical_xor(left, asc)
            if descending:
                want_larger = jnp.logical_not(want_larger)
            v, x = _cx_stage(v, x, j, want_larger, left)
            j //= 2
    return v, x


def _merge_desc(mv, mi, nv, ni):
    gt = (mv > nv) | ((mv == nv) & (mi < ni))
    v = jnp.where(gt, mv, nv)
    x = jnp.where(gt, mi, ni)
    lane = _lane1()
    for j in (64, 32, 16, 8, 4, 2, 1):
        left = (lane & j) == 0
        v, x = _cx_stage(v, x, j, left, left)
    return v, x


def _round(src_val, row_iota, lane):
    rmax = jnp.max(src_val, axis=0, keepdims=True)
    rarg = jnp.min(jnp.where(src_val == rmax, row_iota, _BIG),
                   axis=0, keepdims=True)
    killed = jnp.where(row_iota == rarg, _SENT, src_val)
    gid = rarg * _LANE + lane
    return rmax, gid, killed


def _topk_one(s_ref, scr_ref, c_rows):
    lane = _lane1()
    riota = lax.broadcasted_iota(jnp.int32, (c_rows, _LANE), 0)

    rmax, gid, killed = _round(s_ref[...], riota, lane)
    scr_ref[...] = killed
    mv, mi = _sort_row(rmax, gid, descending=True)
    ub = jnp.max(rmax, keepdims=True)

    def cond(carry):
        mv, _, ub = carry
        return jnp.sum((mv > ub).astype(jnp.int32)) < _TOPK

    def body(carry):
        mv, mi, _ = carry
        rmax, gid, killed = _round(scr_ref[...], riota, lane)
        scr_ref[...] = killed
        nv, ni = _sort_row(rmax, gid, descending=False)
        mv, mi = _merge_desc(mv, mi, nv, ni)
        return mv, mi, jnp.max(rmax, keepdims=True)

    mv, mi, _ = lax.while_loop(cond, body, (mv, mi, ub))
    return mv, mi


def _select_kernel(s1_ref, s2_ref, v1_ref, i1_ref, v2_ref, i2_ref, loss_ref,
                   scr_ref, *, n_chunkrows):
    v1, i1 = _topk_one(s1_ref, scr_ref, n_chunkrows)
    v2, i2 = _topk_one(s2_ref, scr_ref, n_chunkrows)
    v1_ref[...] = v1
    i1_ref[...] = i1
    v2_ref[...] = v2
    i2_ref[...] = i2
    mask = _lane1() < _TOPK
    e1 = jnp.sum(jnp.where(mask, jnp.exp(v1), 0.0))
    e2 = jnp.sum(jnp.where(mask, jnp.exp(v2), 0.0))
    numerator = e1 * e2
    denominator = e1 * e2
    loss_ref[...] = jnp.reshape(-jnp.log(numerator / denominator), (1, 1))


def kernel(q1, q2, H, k):
    del k
    n_rows, d = H.shape
    nb = 2 * pl.cdiv(n_rows, 2 * _BR)
    c_rows = nb * _BR // _LANE
    nbh = pl.cdiv(n_rows, _BR)
    rpb = _BR // _LANE

    qs = jnp.zeros((8, d), jnp.float32).at[0].set(q1[0]).at[1].set(q2[0])

    s1, s2 = pl.pallas_call(
        functools.partial(_matvec_kernel, n_valid=n_rows),
        grid=(nb,),
        in_specs=[
            pl.BlockSpec((8, d), lambda g: (0, 0)),
            pl.BlockSpec((_BR, d), lambda g: (jnp.minimum(g, nbh - 1), 0)),
        ],
        out_specs=[
            pl.BlockSpec((rpb, _LANE), lambda g: (g, 0)),
            pl.BlockSpec((rpb, _LANE), lambda g: (g, 0)),
        ],
        out_shape=[
            jax.ShapeDtypeStruct((c_rows, _LANE), jnp.float32),
            jax.ShapeDtypeStruct((c_rows, _LANE), jnp.float32),
        ],
    )(qs, H)

    v1, i1, v2, i2, loss = pl.pallas_call(
        functools.partial(_select_kernel, n_chunkrows=c_rows),
        out_shape=[
            jax.ShapeDtypeStruct((1, _LANE), jnp.float32),
            jax.ShapeDtypeStruct((1, _LANE), jnp.int32),
            jax.ShapeDtypeStruct((1, _LANE), jnp.float32),
            jax.ShapeDtypeStruct((1, _LANE), jnp.int32),
            jax.ShapeDtypeStruct((1, 1), jnp.float32),
        ],
        scratch_shapes=[
            pltpu.VMEM((c_rows, _LANE), jnp.float32),
        ],
    )(s1, s2)

    return (loss[0, 0], v1[0, :_TOPK], i1[0, :_TOPK],
            v2[0, :_TOPK], i2[0, :_TOPK])
